# baseline (speedup 1.0000x reference)
import functools
import math

import jax
import jax.numpy as jnp
from jax import lax
from jax.experimental import pallas as pl
from jax.experimental.pallas import tpu as pltpu
from jax.experimental.pallas import tpu_sc as plsc

VOCAB_N = 1000000
D = 64
SCALE = math.sqrt(D)

NUM_CORES = 2
NUM_SUBCORES = 16
NW = NUM_CORES * NUM_SUBCORES

B_I = 4096
B_J = 200
PACK = VOCAB_N // 2
NFULL = VOCAB_N // 128
TAIL_V = VOCAB_N - NFULL * 128
N_ITEMS = B_J // 8


def _worker_id():
    return lax.axis_index("s") * NUM_CORES + lax.axis_index("c")


def _conv_body(tabT, packed, stage0, stage1, obuf0, obuf1, staget,
               rsem0, rsem1, wsem0, wsem1):
    wid = _worker_id()
    stage = (stage0, stage1)
    obuf = (obuf0, obuf1)
    rsem = (rsem0, rsem1)
    wsem = (wsem0, wsem1)
    iota = lax.iota(jnp.int32, 16)
    rvecs = [iota + (g % 4) * 16 for g in range(8)]

    def src_slice(b):
        return tabT.at[pl.ds(0, D), pl.ds(b * 128, 128)]

    def dst_slice(b):
        return packed.at[pl.ds(b * 64, 64)]

    def fire_read(b, bb):
        pltpu.async_copy(src_slice(b), stage[bb], rsem[bb])

    def wait_read(b, bb):
        pltpu.make_async_copy(src_slice(b), stage[bb], rsem[bb]).wait()

    def fire_write(b, bb):
        pltpu.async_copy(obuf[bb], dst_slice(b), wsem[bb])

    def wait_write(b, bb):
        pltpu.make_async_copy(obuf[bb], dst_slice(b), wsem[bb]).wait()

    def transpose_block(sref, oref, nrow):
        @plsc.parallel_loop(0, nrow, unroll=8)
        def row_body(l):
            for g in range(8):
                cvec = jnp.full((16,), 1 if g >= 4 else 0, jnp.int32) + 2 * l
                vals = plsc.load_gather(sref, [rvecs[g], cvec])
                oref[l, pl.ds(g * 16, 16)] = vals * SCALE

    def step(t, bb):
        b = wid + 32 * t

        @pl.when(b < NFULL)
        def _():
            wait_read(b, bb)

            @pl.when(t >= 2)
            def _():
                wait_write(b, bb)
            transpose_block(stage[bb], obuf[bb], 64)

            @pl.when(b + 64 < NFULL)
            def _():
                fire_read(b + 64, bb)
            fire_write(b, bb)

    fire_read(wid, 0)

    @pl.when(wid + 32 < NFULL)
    def _():
        fire_read(wid + 32, 1)

    def loop_body(i, c):
        step(2 * i, 0)
        step(2 * i + 1, 1)
        return c

    lax.fori_loop(0, 123, loop_body, 0)
    wait_write(0, 0)
    wait_write(0, 1)

    @pl.when(wid == 31)
    def _():
        pltpu.sync_copy(tabT.at[pl.ds(0, D), pl.ds(NFULL * 128, TAIL_V)],
                        staget)
        @plsc.parallel_loop(0, 32, unroll=8)
        def row_body(l):
            for g in range(8):
                cvec = jnp.full((16,), 1 if g >= 4 else 0, jnp.int32) + 2 * l
                vals = plsc.load_gather(staget, [rvecs[g], cvec])
                obuf0[l, pl.ds(g * 16, 16)] = vals * SCALE
        pltpu.sync_copy(obuf0.at[pl.ds(0, 32)],
                        packed.at[pl.ds(NFULL * 64, 32)])


def _gather_body(xT, packed, outT, xtile, half, and64,
                 rows0, rows1, asm0, asm1, gsem0, gsem1, osem0, osem1):
    wid = _worker_id()
    rows = (rows0, rows1)
    asm = (asm0, asm1)
    gsem = (gsem0, gsem1)
    osem = (osem0, osem1)
    iota = lax.iota(jnp.int32, 16)
    lvecs = [iota + lg * 16 for lg in range(8)]

    def fire_gather(j, bb):
        pltpu.async_copy(packed.at[half.at[j]], rows[bb], gsem[bb])

    def wait_gather(j, bb):
        pltpu.make_async_copy(packed.at[half.at[j]], rows[bb], gsem[bb]).wait()

    def out_slice(jabs):
        return outT.at[jabs, pl.ds(0, D), pl.ds(wid * 128, 128)]

    def item(t, c):
        pltpu.sync_copy(
            xT.at[pl.ds(t * 8, 8), pl.ds(wid * 128, 128)], xtile)
        for j in range(8):
            for g in range(8):
                sl = pl.ds(g * 16, 16)
                v = xtile[j, sl]
                half[j, sl] = lax.shift_right_logical(v, 1)
                and64[j, sl] = (v & 1) * 64

        fire_gather(0, 0)
        for j in range(8):
            bb = j % 2
            if j + 1 < 8:
                fire_gather(j + 1, 1 - bb)
            wait_gather(j, bb)
            if j >= 2:
                pltpu.make_async_copy(asm[bb], out_slice(0), osem[bb]).wait()
            else:
                @pl.when(t > 0)
                def _():
                    pltpu.make_async_copy(asm[bb], out_slice(0),
                                          osem[bb]).wait()
            for lg in range(8):
                avec = and64[j, pl.ds(lg * 16, 16)]
                sl = pl.ds(lg * 16, 16)
                lvec = lvecs[lg]
                rbuf = rows[bb]
                abuf = asm[bb]

                @plsc.parallel_loop(0, D, unroll=8)
                def d_body(d, _avec=avec, _lvec=lvec, _sl=sl,
                           _rbuf=rbuf, _abuf=abuf):
                    vals = plsc.load_gather(_rbuf, [_lvec, _avec + d])
                    _abuf[d, _sl] = vals
            pltpu.async_copy(asm[bb], out_slice(t * 8 + j), osem[bb])
        return c

    lax.fori_loop(0, N_ITEMS, item, 0)
    pltpu.make_async_copy(asm0, out_slice(0), osem0).wait()
    pltpu.make_async_copy(asm1, out_slice(0), osem1).wait()


@jax.jit
def kernel(x, table):
    xT = x.T.astype(jnp.int32)
    tableT = table.T
    mesh = plsc.VectorSubcoreMesh(
        core_axis_name="c", subcore_axis_name="s",
        num_cores=NUM_CORES, num_subcores=NUM_SUBCORES,
    )
    cp = pltpu.CompilerParams(use_tc_tiling_on_sc=True,
                              needs_layout_passes=False)
    packed = pl.kernel(
        _conv_body,
        out_type=jax.ShapeDtypeStruct((PACK, 128), jnp.float32),
        mesh=mesh,
        scratch_types=[
            pltpu.VMEM((D, 128), jnp.float32),
            pltpu.VMEM((D, 128), jnp.float32),
            pltpu.VMEM((D, 128), jnp.float32),
            pltpu.VMEM((D, 128), jnp.float32),
            pltpu.VMEM((D, TAIL_V), jnp.float32),
            pltpu.SemaphoreType.DMA,
            pltpu.SemaphoreType.DMA,
            pltpu.SemaphoreType.DMA,
            pltpu.SemaphoreType.DMA,
        ],
        compiler_params=cp,
    )(tableT)
    outT = pl.kernel(
        _gather_body,
        out_type=jax.ShapeDtypeStruct((B_J, D, B_I), jnp.float32),
        mesh=mesh,
        scratch_types=[
            pltpu.VMEM((8, 128), jnp.int32),
            pltpu.VMEM((8, 128), jnp.int32),
            pltpu.VMEM((8, 128), jnp.int32),
            pltpu.VMEM((128, 128), jnp.float32),
            pltpu.VMEM((128, 128), jnp.float32),
            pltpu.VMEM((D, 128), jnp.float32),
            pltpu.VMEM((D, 128), jnp.float32),
            pltpu.SemaphoreType.DMA,
            pltpu.SemaphoreType.DMA,
            pltpu.SemaphoreType.DMA,
            pltpu.SemaphoreType.DMA,
        ],
        compiler_params=cp,
    )(xT, packed)
    return outT.transpose(2, 0, 1)


# speedup vs baseline: 1.2263x; 1.2263x over previous
import functools
import math

import jax
import jax.numpy as jnp
from jax import lax
from jax.experimental import pallas as pl
from jax.experimental.pallas import tpu as pltpu
from jax.experimental.pallas import tpu_sc as plsc

VOCAB_N = 1000000
D = 64
SCALE = math.sqrt(D)

NUM_CORES = 2
NUM_SUBCORES = 16
NW = NUM_CORES * NUM_SUBCORES

TOTAL = 4096 * 200
PER_W = TOTAL // NW
CHUNK = 512
K = CHUNK // 128
N_CHUNKS = PER_W // CHUNK


def _body(x_hbm, table_hbm, out_hbm,
          idx0, idx1, rows0, rows1, gsem0, gsem1, ssem0, ssem1):
    wid = lax.axis_index("s") * NUM_CORES + lax.axis_index("c")
    chunk0 = wid * N_CHUNKS
    out0 = wid * PER_W

    idx = (idx0, idx1)
    rows = (rows0, rows1)
    gsem = (gsem0, gsem1)
    ssem = (ssem0, ssem1)

    def fire_gather(g, b):
        pltpu.sync_copy(x_hbm.at[chunk0 + g], idx[b])
        for j in range(K):
            pltpu.async_copy(
                table_hbm.at[idx[b].at[j]],
                rows[b].at[pl.ds(j * 128, 128)],
                gsem[b],
            )

    def wait_gather(b):
        for j in range(K):
            pltpu.make_async_copy(
                table_hbm.at[idx[b].at[j]],
                rows[b].at[pl.ds(j * 128, 128)],
                gsem[b],
            ).wait()

    def out_slice(g):
        return out_hbm.at[pl.ds(out0 + g * CHUNK, CHUNK)]

    def scale_buf(b):
        def scale_row(i, c):
            for j in range(D // 16):
                sl = pl.ds(j * 16, 16)
                rows[b][i, sl] = rows[b][i, sl] * SCALE
            return c
        lax.fori_loop(0, CHUNK, scale_row, 0, unroll=8)

    def step(g, b):
        @pl.when(g > 0)
        def _():
            pltpu.make_async_copy(rows[1 - b], out_slice(g - 1), ssem[1 - b]).wait()

        @pl.when(g + 1 < N_CHUNKS)
        def _():
            fire_gather(g + 1, 1 - b)

        wait_gather(b)
        scale_buf(b)
        pltpu.async_copy(rows[b], out_slice(g), ssem[b])

    fire_gather(0, 0)

    def loop_body(i, c):
        step(2 * i, 0)
        step(2 * i + 1, 1)
        return c

    lax.fori_loop(0, N_CHUNKS // 2, loop_body, 0)

    pltpu.make_async_copy(rows[1], out_slice(N_CHUNKS - 1), ssem[1]).wait()


@functools.partial(jax.jit, donate_argnums=())
def kernel(x, table):
    x2 = x.reshape(NW * N_CHUNKS, K, 128).astype(jnp.int32)
    mesh = plsc.VectorSubcoreMesh(
        core_axis_name="c", subcore_axis_name="s",
        num_cores=NUM_CORES, num_subcores=NUM_SUBCORES,
    )
    run = pl.kernel(
        _body,
        out_type=jax.ShapeDtypeStruct((TOTAL, D), jnp.float32),
        mesh=mesh,
        scratch_types=[
            pltpu.VMEM((K, 128), jnp.int32),
            pltpu.VMEM((K, 128), jnp.int32),
            pltpu.VMEM((CHUNK, D), jnp.float32),
            pltpu.VMEM((CHUNK, D), jnp.float32),
            pltpu.SemaphoreType.DMA,
            pltpu.SemaphoreType.DMA,
            pltpu.SemaphoreType.DMA,
            pltpu.SemaphoreType.DMA,
        ],
        compiler_params=pltpu.CompilerParams(use_tc_tiling_on_sc=False),
    )
    out = run(x2, table)
    return out.reshape(x.shape[0], x.shape[1], D)
